# baseline (speedup 1.0000x reference)
import functools

import jax
import jax.numpy as jnp
from jax import lax
from jax.experimental import pallas as pl
from jax.experimental.pallas import tpu as pltpu
from jax.experimental.pallas import tpu_sc as plsc

D = 128
L = 16
NC = 2
NS = 16
NW = NC * NS
K = 128

N_B = 10000
N_U = 50000

RA = 320
NB_PAD = NW * RA
RB = 224
CB = 7
NU_PAD = NW * CB * RB
EPAD = 4 * K


def _segmean_kernel(nrows, nchunks):
    mesh = plsc.VectorSubcoreMesh(core_axis_name="c", subcore_axis_name="s")
    out_rows = NW * nchunks * nrows

    @functools.partial(
        pl.kernel,
        mesh=mesh,
        out_type=jax.ShapeDtypeStruct((out_rows, D), jnp.float32),
        scratch_types=[
            pltpu.VMEM((16,), jnp.int32),
            pltpu.VMEM((2, K), jnp.int32),
            pltpu.VMEM((2, K), jnp.int32),
            pltpu.VMEM((2, K, D), jnp.float32),
            pltpu.VMEM((nrows + 1, D), jnp.float32),
            pltpu.VMEM((nrows + 16,), jnp.float32),
            pltpu.VMEM((K,), jnp.int32),
            pltpu.SemaphoreType.DMA,
            pltpu.SemaphoreType.DMA,
        ],
    )
    def seg_kernel(table_hbm, row_hbm, col_hbm, meta_hbm, out_hbm,
                   meta_v, idxr, idxc, gbuf, acc, deg, locbuf, sem_i, sem_g):
        wid = lax.axis_index("s") * NC + lax.axis_index("c")
        zero = jnp.zeros((L,), jnp.float32)
        onehot = jnp.where(lax.iota(jnp.int32, L) == 0, 1.0, 0.0)

        def do_chunk(ch, _):
            cid = wid * nchunks + ch
            r0 = cid * nrows
            pltpu.sync_copy(meta_hbm.at[cid], meta_v)
            mv = meta_v[...]
            e0 = mv[0]
            nb = mv[1]

            def zero_body(r, _):
                for c in range(D // L):
                    acc[r, pl.ds(c * L, L)] = zero
                return 0
            lax.fori_loop(0, nrows + 1, zero_body, 0)

            def zero_deg(g, _):
                deg[pl.ds(g * L, L)] = zero
                return 0
            lax.fori_loop(0, (nrows + 16) // L, zero_deg, 0)

            def issue_idx(bb, slot):
                s = pl.multiple_of(e0 + bb * K, 8)
                pltpu.async_copy(row_hbm.at[pl.ds(s, K)], idxr.at[slot], sem_i)
                pltpu.async_copy(col_hbm.at[pl.ds(s, K)], idxc.at[slot], sem_i)

            def wait_idx():
                pltpu.make_async_copy(row_hbm.at[pl.ds(0, K)], idxr.at[0],
                                      sem_i).wait()
                pltpu.make_async_copy(col_hbm.at[pl.ds(0, K)], idxc.at[0],
                                      sem_i).wait()

            def issue_gather(slot):
                pltpu.async_copy(table_hbm.at[idxc.at[slot]], gbuf.at[slot],
                                 sem_g)

            def wait_gather():
                pltpu.make_async_copy(table_hbm.at[idxc.at[0]], gbuf.at[0],
                                      sem_g).wait()

            issue_idx(0, 0)
            issue_idx(1, 1)
            wait_idx()
            issue_gather(0)

            def batch_body(b, _):
                par = b & 1
                npar = (b + 1) & 1

                @pl.when(b + 1 < nb)
                def _():
                    wait_idx()
                wait_gather()

                rowb = idxr.at[par]
                for g in range(K // L):
                    rows = rowb[pl.ds(g * L, L)]
                    locv = rows - r0
                    okv = (locv >= 0) & (locv < nrows)
                    locbuf[pl.ds(g * L, L)] = jnp.where(okv, locv, nrows)

                @pl.when(b + 2 < nb)
                def _():
                    issue_idx(b + 2, par)

                @pl.when(b + 1 < nb)
                def _():
                    issue_gather(npar)

                gb = gbuf.at[par]

                @plsc.parallel_loop(0, K // L, unroll=K // L)
                def _(g):
                    locv = locbuf[pl.ds(g * L, L)]
                    for j in range(L):
                        loc = locv[j]
                        e = g * L + j
                        vals = [gb[e, pl.ds(c * L, L)] for c in range(D // L)]
                        for c in range(D // L):
                            plsc.addupdate(acc.at[loc, pl.ds(c * L, L)],
                                           vals[c])
                        plsc.addupdate(deg.at[pl.ds(loc, L)], onehot)
                return 0
            lax.fori_loop(0, nb, batch_body, 0)

            def norm_body(g, _):
                dg = jnp.maximum(deg[pl.ds(g * L, L)], 1.0)
                inv = 1.0 / dg
                for j in range(L):
                    r = g * L + j
                    f = inv[j]
                    for c in range(D // L):
                        acc[r, pl.ds(c * L, L)] = acc[r, pl.ds(c * L, L)] * f
                return 0
            lax.fori_loop(0, nrows // L, norm_body, 0)

            pltpu.sync_copy(acc.at[pl.ds(0, nrows)],
                            out_hbm.at[pl.ds(r0, nrows)])
            return 0

        lax.fori_loop(0, nchunks, do_chunk, 0)

    return seg_kernel


def _chunk_meta(row_sorted_padded, nrows, nchunks):
    starts = jnp.arange(NW * nchunks, dtype=jnp.int32) * nrows
    lo = jnp.searchsorted(row_sorted_padded, starts, side="left").astype(jnp.int32)
    hi = jnp.searchsorted(row_sorted_padded, starts + nrows,
                          side="left").astype(jnp.int32)
    e0 = lo & ~7
    nb = jnp.maximum((hi - e0 + K - 1) // K, 2)
    meta = jnp.zeros((NW * nchunks, 16), jnp.int32)
    return meta.at[:, 0].set(e0).at[:, 1].set(nb)


def kernel(user_emb, item_emb, hv_row, hv_col, hu_row, hu_col):
    del user_emb
    hv_row_p = jnp.concatenate([hv_row, jnp.full((EPAD,), NB_PAD, jnp.int32)])
    hv_col_p = jnp.concatenate([hv_col, jnp.zeros((EPAD,), jnp.int32)])
    hu_row_p = jnp.concatenate([hu_row, jnp.full((EPAD,), NU_PAD, jnp.int32)])
    hu_col_p = jnp.concatenate([hu_col, jnp.zeros((EPAD,), jnp.int32)])
    meta_a = _chunk_meta(hv_row_p, RA, 1)
    meta_b = _chunk_meta(hu_row_p, RB, CB)
    bf = _segmean_kernel(RA, 1)(item_emb, hv_row_p, hv_col_p, meta_a)
    ulv = _segmean_kernel(RB, CB)(bf, hu_row_p, hu_col_p, meta_b)
    return ulv[:N_U]


# speedup vs baseline: 1.1520x; 1.1520x over previous
import functools

import jax
import jax.numpy as jnp
from jax import lax
from jax.experimental import pallas as pl
from jax.experimental.pallas import tpu as pltpu
from jax.experimental.pallas import tpu_sc as plsc

D = 128
W = D // 2
L = 16
NC = 2
NS = 16
NW = NC * NS
K = 128

N_B = 10000
N_U = 50000

RA = 320
NB_PAD = NW * RA
RB = 224
CB = 7
NU_PAD = NW * CB * RB
EPAD = 4 * K

HIMASK = -65536
RND = 0x8000


def _segmean_kernel(nrows, nchunks, packed_out):
    mesh = plsc.VectorSubcoreMesh(core_axis_name="c", subcore_axis_name="s")
    out_rows = NW * nchunks * nrows
    if packed_out:
        out_type = jax.ShapeDtypeStruct((out_rows, W), jnp.int32)
    else:
        out_type = jax.ShapeDtypeStruct((out_rows, D), jnp.float32)

    @functools.partial(
        pl.kernel,
        mesh=mesh,
        out_type=out_type,
        compiler_params=pltpu.CompilerParams(use_tc_tiling_on_sc=False),
        scratch_types=[
            pltpu.VMEM((16,), jnp.int32),
            pltpu.VMEM((2, K), jnp.int32),
            pltpu.VMEM((2, K), jnp.int32),
            pltpu.VMEM((2, K, W), jnp.int32),
            pltpu.VMEM((nrows + 1, D), jnp.float32),
            pltpu.VMEM((nrows + 16,), jnp.float32),
            pltpu.VMEM((K,), jnp.int32),
            pltpu.VMEM((nrows, W), jnp.int32),
            pltpu.SemaphoreType.DMA,
            pltpu.SemaphoreType.DMA,
        ],
    )
    def seg_kernel(table_hbm, row_hbm, col_hbm, meta_hbm, out_hbm,
                   meta_v, idxr, idxc, gbuf, acc, deg, locbuf, obuf,
                   sem_i, sem_g):
        wid = lax.axis_index("s") * NC + lax.axis_index("c")
        zero = jnp.zeros((L,), jnp.float32)
        onehot = jnp.where(lax.iota(jnp.int32, L) == 0, 1.0, 0.0)

        def do_chunk(ch, _):
            cid = wid * nchunks + ch
            r0 = cid * nrows
            pltpu.sync_copy(meta_hbm.at[cid], meta_v)
            mv = meta_v[...]
            e0 = mv[0]
            nb = mv[1]

            def zero_body(r, _):
                for c in range(D // L):
                    acc[r, pl.ds(c * L, L)] = zero
                return 0
            lax.fori_loop(0, nrows + 1, zero_body, 0)

            def zero_deg(g, _):
                deg[pl.ds(g * L, L)] = zero
                return 0
            lax.fori_loop(0, (nrows + 16) // L, zero_deg, 0)

            def issue_idx(bb, slot):
                s = pl.multiple_of(e0 + bb * K, 8)
                pltpu.async_copy(row_hbm.at[pl.ds(s, K)], idxr.at[slot], sem_i)
                pltpu.async_copy(col_hbm.at[pl.ds(s, K)], idxc.at[slot], sem_i)

            def wait_idx():
                pltpu.make_async_copy(row_hbm.at[pl.ds(0, K)], idxr.at[0],
                                      sem_i).wait()
                pltpu.make_async_copy(col_hbm.at[pl.ds(0, K)], idxc.at[0],
                                      sem_i).wait()

            def issue_gather(slot):
                pltpu.async_copy(table_hbm.at[idxc.at[slot]], gbuf.at[slot],
                                 sem_g)

            def wait_gather():
                pltpu.make_async_copy(table_hbm.at[idxc.at[0]], gbuf.at[0],
                                      sem_g).wait()

            issue_idx(0, 0)
            issue_idx(1, 1)
            wait_idx()
            issue_gather(0)

            def batch_body(b, _):
                par = b & 1
                npar = (b + 1) & 1

                @pl.when(b + 1 < nb)
                def _():
                    wait_idx()
                wait_gather()

                rowb = idxr.at[par]
                for g in range(K // L):
                    rows = rowb[pl.ds(g * L, L)]
                    locv = rows - r0
                    okv = (locv >= 0) & (locv < nrows)
                    locbuf[pl.ds(g * L, L)] = jnp.where(okv, locv, nrows)

                @pl.when(b + 2 < nb)
                def _():
                    issue_idx(b + 2, par)

                @pl.when(b + 1 < nb)
                def _():
                    issue_gather(npar)

                gb = gbuf.at[par]

                @plsc.parallel_loop(0, K // L, unroll=K // L)
                def _(g):
                    locv = locbuf[pl.ds(g * L, L)]
                    for j in range(L):
                        loc = locv[j]
                        e = g * L + j
                        vals = []
                        for k in range(D // 32):
                            w = gb[e, pl.ds(16 * k, L)]
                            a = lax.bitcast_convert_type(w << 16, jnp.float32)
                            b2 = lax.bitcast_convert_type(w & HIMASK, jnp.float32)
                            vals += [a, b2]
                        for c in range(D // L):
                            plsc.addupdate(acc.at[loc, pl.ds(c * L, L)],
                                           vals[c])
                        plsc.addupdate(deg.at[pl.ds(loc, L)], onehot)
                return 0
            lax.fori_loop(0, nb, batch_body, 0)

            def norm_body(g, _):
                dg = jnp.maximum(deg[pl.ds(g * L, L)], 1.0)
                inv = 1.0 / dg
                for j in range(L):
                    r = g * L + j
                    f = inv[j]
                    if packed_out:
                        for k in range(D // 32):
                            a = acc[r, pl.ds(32 * k, L)] * f
                            b2 = acc[r, pl.ds(32 * k + L, L)] * f
                            ab = lax.bitcast_convert_type(a, jnp.int32) + RND
                            bb = lax.bitcast_convert_type(b2, jnp.int32) + RND
                            obuf[r, pl.ds(16 * k, L)] = (
                                lax.shift_right_logical(ab, 16)
                                | (bb & HIMASK))
                    else:
                        for c in range(D // L):
                            acc[r, pl.ds(c * L, L)] = (
                                acc[r, pl.ds(c * L, L)] * f)
                return 0
            lax.fori_loop(0, nrows // L, norm_body, 0)

            if packed_out:
                pltpu.sync_copy(obuf.at[pl.ds(0, nrows)],
                                out_hbm.at[pl.ds(r0, nrows)])
            else:
                pltpu.sync_copy(acc.at[pl.ds(0, nrows)],
                                out_hbm.at[pl.ds(r0, nrows)])
            return 0

        lax.fori_loop(0, nchunks, do_chunk, 0)

    return seg_kernel


def _chunk_meta(row_sorted_padded, nrows, nchunks):
    starts = jnp.arange(NW * nchunks, dtype=jnp.int32) * nrows
    lo = jnp.searchsorted(row_sorted_padded, starts, side="left").astype(jnp.int32)
    hi = jnp.searchsorted(row_sorted_padded, starts + nrows,
                          side="left").astype(jnp.int32)
    e0 = lo & ~7
    nb = jnp.maximum((hi - e0 + K - 1) // K, 2)
    meta = jnp.zeros((NW * nchunks, 16), jnp.int32)
    return meta.at[:, 0].set(e0).at[:, 1].set(nb)


def _pack_table(x):
    n = x.shape[0]
    xb = x.astype(jnp.bfloat16).reshape(n, D // 32, 2, L)
    xb = jnp.swapaxes(xb, 2, 3)
    return lax.bitcast_convert_type(
        xb.reshape(n, W, 2), jnp.int32)


def kernel(user_emb, item_emb, hv_row, hv_col, hu_row, hu_col):
    del user_emb
    hv_row_p = jnp.concatenate([hv_row, jnp.full((EPAD,), NB_PAD, jnp.int32)])
    hv_col_p = jnp.concatenate([hv_col, jnp.zeros((EPAD,), jnp.int32)])
    hu_row_p = jnp.concatenate([hu_row, jnp.full((EPAD,), NU_PAD, jnp.int32)])
    hu_col_p = jnp.concatenate([hu_col, jnp.zeros((EPAD,), jnp.int32)])
    meta_a = _chunk_meta(hv_row_p, RA, 1)
    meta_b = _chunk_meta(hu_row_p, RB, CB)
    it_pk = _pack_table(item_emb)
    bf_pk = _segmean_kernel(RA, 1, True)(it_pk, hv_row_p, hv_col_p, meta_a)
    ulv = _segmean_kernel(RB, CB, False)(bf_pk, hu_row_p, hu_col_p, meta_b)
    return ulv[:N_U]
